# baseline (speedup 1.0000x reference)
import functools

import jax
import jax.numpy as jnp
from jax import lax
from jax.experimental import pallas as pl
from jax.experimental.pallas import tpu as pltpu
from jax.experimental.pallas import tpu_sc as plsc

_B = 16384
_D = 128
_LANES = 16
_NC = 2
_NS = 16
_NW = _NC * _NS
_ROWS_PER_W = _B // _NW
_SIZES = (32, 96, 128, 128, 96, 32)
_OFFS = tuple(sum(_SIZES[:i]) for i in range(len(_SIZES)))
_NCHUNK = len(_SIZES)
assert sum(_SIZES) == _ROWS_PER_W


def _sc_gather_mult(in_repr, group_idx, lambdas):
    mesh = plsc.VectorSubcoreMesh(core_axis_name="c", subcore_axis_name="s")

    lam_scratch = [pltpu.VMEM((s, _D), jnp.float32) for s in _SIZES]
    x_scratch = [pltpu.VMEM((max(_SIZES), _D), jnp.float32) for _ in range(2)]
    sems = [pltpu.SemaphoreType.DMA for _ in range(2 * _NCHUNK + 2)]

    @functools.partial(
        pl.kernel,
        out_type=jax.ShapeDtypeStruct((_B, _D), jnp.float32),
        mesh=mesh,
        scratch_types=(
            [pltpu.VMEM((_ROWS_PER_W,), jnp.int32)]
            + lam_scratch + x_scratch + sems
        ),
    )
    def k(in_hbm, idx_hbm, lam_hbm, out_hbm, idx_v, *bufs):
        lam = list(bufs[:_NCHUNK])
        xb = list(bufs[_NCHUNK:_NCHUNK + 2])
        gsem = list(bufs[_NCHUNK + 2:2 * _NCHUNK + 2])
        xsem = list(bufs[2 * _NCHUNK + 2:2 * _NCHUNK + 4])
        osem = list(bufs[2 * _NCHUNK + 4:])

        wid = lax.axis_index("s") * _NC + lax.axis_index("c")
        base = wid * _ROWS_PER_W

        xgets = [None] * _NCHUNK
        puts = [None] * _NCHUNK

        def start_x(c):
            xgets[c] = pltpu.async_copy(
                in_hbm.at[pl.ds(base + _OFFS[c], _SIZES[c])],
                xb[c % 2].at[pl.ds(0, _SIZES[c])],
                xsem[c % 2],
            )

        start_x(0)
        start_x(1)

        def start_gather(c):
            return pltpu.async_copy(
                lam_hbm.at[idx_v.at[pl.ds(_OFFS[c], _SIZES[c])]], lam[c], gsem[c]
            )

        head = [c for c in range(_NCHUNK) if _OFFS[c] + _SIZES[c] <= 128]
        rest = [c for c in range(_NCHUNK) if c not in head]
        gets = [None] * _NCHUNK
        pltpu.sync_copy(idx_hbm.at[pl.ds(base, 128)], idx_v.at[pl.ds(0, 128)])
        for c in head:
            gets[c] = start_gather(c)
        pltpu.sync_copy(
            idx_hbm.at[pl.ds(base + 128, _ROWS_PER_W - 128)],
            idx_v.at[pl.ds(128, _ROWS_PER_W - 128)],
        )
        for c in rest:
            gets[c] = start_gather(c)
        for c in range(_NCHUNK):
            xv = xb[c % 2]
            gets[c].wait()
            xgets[c].wait()

            @pl.loop(0, _SIZES[c])
            def _(r):
                for c0 in range(0, _D, _LANES):
                    lam[c][r, pl.ds(c0, _LANES)] = (
                        lam[c][r, pl.ds(c0, _LANES)] * xv[r, pl.ds(c0, _LANES)]
                    )

            puts[c] = pltpu.async_copy(
                lam[c], out_hbm.at[pl.ds(base + _OFFS[c], _SIZES[c])], osem[c % 2]
            )
            if c + 2 < _NCHUNK:
                start_x(c + 2)
        for c in range(_NCHUNK):
            puts[c].wait()

    return k(in_repr, group_idx, lambdas)


def kernel(in_repr, group_idx, lambdas):
    return _sc_gather_mult(in_repr, group_idx.astype(jnp.int32), lambdas)


# speedup vs baseline: 1.0054x; 1.0054x over previous
import functools

import jax
import jax.numpy as jnp
from jax import lax
from jax.experimental import pallas as pl
from jax.experimental.pallas import tpu as pltpu
from jax.experimental.pallas import tpu_sc as plsc

_B = 16384
_D = 128
_LANES = 16
_NC = 2
_NS = 16
_NW = _NC * _NS
_ROWS_PER_W = _B // _NW
_SIZES = (64, 128, 128, 128, 64)
_OFFS = tuple(sum(_SIZES[:i]) for i in range(len(_SIZES)))
_NCHUNK = len(_SIZES)
assert sum(_SIZES) == _ROWS_PER_W


def _sc_gather_mult(in_repr, group_idx, lambdas):
    mesh = plsc.VectorSubcoreMesh(core_axis_name="c", subcore_axis_name="s")

    lam_scratch = [pltpu.VMEM((s, _D), jnp.float32) for s in _SIZES]
    x_scratch = [pltpu.VMEM((max(_SIZES), _D), jnp.float32) for _ in range(2)]
    sems = [pltpu.SemaphoreType.DMA for _ in range(2 * _NCHUNK + 2)]

    @functools.partial(
        pl.kernel,
        out_type=jax.ShapeDtypeStruct((_B, _D), jnp.float32),
        mesh=mesh,
        scratch_types=(
            [pltpu.VMEM((_ROWS_PER_W,), jnp.int32)]
            + lam_scratch + x_scratch + sems
        ),
    )
    def k(in_hbm, idx_hbm, lam_hbm, out_hbm, idx_v, *bufs):
        lam = list(bufs[:_NCHUNK])
        xb = list(bufs[_NCHUNK:_NCHUNK + 2])
        gsem = list(bufs[_NCHUNK + 2:2 * _NCHUNK + 2])
        xsem = list(bufs[2 * _NCHUNK + 2:2 * _NCHUNK + 4])
        osem = list(bufs[2 * _NCHUNK + 4:])

        wid = lax.axis_index("s") * _NC + lax.axis_index("c")
        base = wid * _ROWS_PER_W

        xgets = [None] * _NCHUNK
        puts = [None] * _NCHUNK

        def start_x(c):
            xgets[c] = pltpu.async_copy(
                in_hbm.at[pl.ds(base + _OFFS[c], _SIZES[c])],
                xb[c % 2].at[pl.ds(0, _SIZES[c])],
                xsem[c % 2],
            )

        start_x(0)
        start_x(1)

        def start_gather(c):
            return pltpu.async_copy(
                lam_hbm.at[idx_v.at[pl.ds(_OFFS[c], _SIZES[c])]], lam[c], gsem[c]
            )

        gets = [None] * _NCHUNK
        pltpu.sync_copy(
            idx_hbm.at[pl.ds(base, _SIZES[0])], idx_v.at[pl.ds(0, _SIZES[0])]
        )
        gets[0] = start_gather(0)
        pltpu.sync_copy(
            idx_hbm.at[pl.ds(base + _SIZES[0], _ROWS_PER_W - _SIZES[0])],
            idx_v.at[pl.ds(_SIZES[0], _ROWS_PER_W - _SIZES[0])],
        )
        for c in range(1, _NCHUNK):
            gets[c] = start_gather(c)
        for c in range(_NCHUNK):
            xv = xb[c % 2]
            gets[c].wait()
            xgets[c].wait()

            @pl.loop(0, _SIZES[c])
            def _(r):
                for c0 in range(0, _D, _LANES):
                    lam[c][r, pl.ds(c0, _LANES)] = (
                        lam[c][r, pl.ds(c0, _LANES)] * xv[r, pl.ds(c0, _LANES)]
                    )

            puts[c] = pltpu.async_copy(
                lam[c], out_hbm.at[pl.ds(base + _OFFS[c], _SIZES[c])], osem[c % 2]
            )
            if c + 2 < _NCHUNK:
                start_x(c + 2)
        for c in range(_NCHUNK):
            puts[c].wait()

    return k(in_repr, group_idx, lambdas)


def kernel(in_repr, group_idx, lambdas):
    return _sc_gather_mult(in_repr, group_idx.astype(jnp.int32), lambdas)


# speedup vs baseline: 1.0329x; 1.0274x over previous
import functools

import jax
import jax.numpy as jnp
from jax import lax
from jax.experimental import pallas as pl
from jax.experimental.pallas import tpu as pltpu
from jax.experimental.pallas import tpu_sc as plsc

_B = 16384
_D = 128
_LANES = 16
_NC = 2
_NS = 16
_NW = _NC * _NS
_ROWS_PER_W = _B // _NW
_SIZES = (64, 128, 128, 128, 64)
_OFFS = tuple(sum(_SIZES[:i]) for i in range(len(_SIZES)))
_NCHUNK = len(_SIZES)
assert sum(_SIZES) == _ROWS_PER_W


def _sc_gather_mult(in_repr, group_idx, lambdas):
    mesh = plsc.VectorSubcoreMesh(core_axis_name="c", subcore_axis_name="s")

    lam_scratch = [pltpu.VMEM((s, _D), jnp.float32) for s in _SIZES]
    x_scratch = [pltpu.VMEM((max(_SIZES), _D), jnp.float32) for _ in range(2)]
    sems = [pltpu.SemaphoreType.DMA for _ in range(2 * _NCHUNK + 2)]

    @functools.partial(
        pl.kernel,
        out_type=jax.ShapeDtypeStruct((_B, _D), jnp.float32),
        mesh=mesh,
        scratch_types=(
            [pltpu.VMEM((_ROWS_PER_W,), jnp.int32)]
            + lam_scratch + x_scratch + sems
        ),
    )
    def k(in_hbm, idx_hbm, lam_hbm, out_hbm, idx_v, *bufs):
        lam = list(bufs[:_NCHUNK])
        xb = list(bufs[_NCHUNK:_NCHUNK + 2])
        gsem = list(bufs[_NCHUNK + 2:2 * _NCHUNK + 2])
        xsem = list(bufs[2 * _NCHUNK + 2:2 * _NCHUNK + 4])
        osem = list(bufs[2 * _NCHUNK + 4:])

        wid = lax.axis_index("s") * _NC + lax.axis_index("c")
        base = wid * _ROWS_PER_W

        xgets = [None] * _NCHUNK
        puts = [None] * _NCHUNK

        def start_x(c):
            xgets[c] = pltpu.async_copy(
                in_hbm.at[pl.ds(base + _OFFS[c], _SIZES[c])],
                xb[c % 2].at[pl.ds(0, _SIZES[c])],
                xsem[c % 2],
            )

        start_x(0)
        start_x(1)

        def start_gather(c):
            return pltpu.async_copy(
                lam_hbm.at[idx_v.at[pl.ds(_OFFS[c], _SIZES[c])]], lam[c], gsem[c]
            )

        pltpu.sync_copy(idx_hbm.at[pl.ds(base, _ROWS_PER_W)], idx_v)
        gets = [start_gather(c) for c in range(_NCHUNK)]
        for c in range(_NCHUNK):
            xv = xb[c % 2]
            gets[c].wait()
            xgets[c].wait()

            @pl.loop(0, _SIZES[c])
            def _(r):
                for c0 in range(0, _D, _LANES):
                    lam[c][r, pl.ds(c0, _LANES)] = (
                        lam[c][r, pl.ds(c0, _LANES)] * xv[r, pl.ds(c0, _LANES)]
                    )

            puts[c] = pltpu.async_copy(
                lam[c], out_hbm.at[pl.ds(base + _OFFS[c], _SIZES[c])], osem[c % 2]
            )
            if c + 2 < _NCHUNK:
                start_x(c + 2)
        for c in range(_NCHUNK):
            puts[c].wait()

    return k(in_repr, group_idx, lambdas)


def kernel(in_repr, group_idx, lambdas):
    return _sc_gather_mult(in_repr, group_idx.astype(jnp.int32), lambdas)
